# baseline (speedup 1.0000x reference)
import functools

import jax
import jax.numpy as jnp
from jax import lax
from jax.experimental import pallas as pl
from jax.experimental.pallas import tpu as pltpu
from jax.experimental.pallas import tpu_sc as plsc

E = 8
K = 2
B = 32
D = 4096
F = 2048

FB = 256
NF = F // FB

L = 16
NC = 2
NS = 16

_NEG = -1e30


def _sc_routing_body(x_hbm, g_hbm, out_hbm, xv, gv, sv, wv, sem):
    b = lax.axis_index("s") * NC + lax.axis_index("c")
    pltpu.sync_copy(x_hbm.at[b], xv)
    pltpu.sync_copy(g_hbm, gv)

    lane = lax.broadcasted_iota(jnp.int32, (L,), 0)
    pair = jnp.where(lane >= 8, 1, 0)

    def body(j, carry):
        a0, a1, a2, a3, idxv = carry
        a0 += plsc.load_gather(xv, [idxv]) * gv[pl.ds(4 * j * L, L)]
        a1 += plsc.load_gather(xv, [idxv + 2]) * gv[pl.ds((4 * j + 1) * L, L)]
        a2 += plsc.load_gather(xv, [idxv + 4]) * gv[pl.ds((4 * j + 2) * L, L)]
        a3 += plsc.load_gather(xv, [idxv + 6]) * gv[pl.ds((4 * j + 3) * L, L)]
        return (a0, a1, a2, a3, idxv + 8)

    z = jnp.zeros((L,), jnp.float32)
    a0, a1, a2, a3, _ = lax.fori_loop(0, D // 8, body, (z, z, z, z, pair),
                                      unroll=4)
    acc = (a0 + a1) + (a2 + a3)

    sv[pl.ds(0, L)] = acc
    sv[pl.ds(L, L)] = jnp.zeros((L,), jnp.float32)
    lv = acc + sv[pl.ds(8, L)]
    lv = jnp.where(lane < E, lv, _NEG)
    m1 = jnp.broadcast_to(jnp.max(lv, axis=0), (L,))
    i1 = jnp.broadcast_to(jnp.min(jnp.where(lv == m1, lane, L), axis=0), (L,))
    lv2 = jnp.where(lane == i1, _NEG, lv)
    m2 = jnp.broadcast_to(jnp.max(lv2, axis=0), (L,))
    i2 = jnp.broadcast_to(jnp.min(jnp.where(lv2 == m2, lane, L), axis=0), (L,))

    t = jnp.exp(m2 - m1)
    p1 = 1.0 / (1.0 + t)
    p2 = t / (1.0 + t)
    wv[...] = jnp.where(lane == i1, p1, 0.0) + jnp.where(lane == i2, p2, 0.0)
    pltpu.sync_copy(wv, out_hbm.at[b])


@functools.partial(
    pl.kernel,
    out_type=jax.ShapeDtypeStruct((B, L), jnp.float32),
    mesh=plsc.VectorSubcoreMesh(core_axis_name="c", subcore_axis_name="s"),
    compiler_params=pltpu.CompilerParams(needs_layout_passes=False),
    scratch_types=[
        pltpu.VMEM((D,), jnp.float32),
        pltpu.VMEM((D // 2 * L,), jnp.float32),
        pltpu.VMEM((2 * L,), jnp.float32),
        pltpu.VMEM((L,), jnp.float32),
        pltpu.SemaphoreType.DMA,
    ],
)
def _sc_routing(x_hbm, g_hbm, out_hbm, xv, gv, sv, wv, sem):
    _sc_routing_body(x_hbm, g_hbm, out_hbm, xv, gv, sv, wv, sem)


def _moe_body(x_ref, we_ref, w1_ref, w3_ref, w2_ref, out_ref, acc_ref):
    e = pl.program_id(0)
    f = pl.program_id(1)

    @pl.when((e == 0) & (f == 0))
    def _init():
        out_ref[...] = jnp.zeros_like(out_ref)

    @pl.when(f == 0)
    def _zero_acc():
        acc_ref[...] = jnp.zeros_like(acc_ref)

    x = x_ref[...].astype(jnp.bfloat16)
    w1b = w1_ref[0].astype(jnp.bfloat16)
    w3b = w3_ref[0].astype(jnp.bfloat16)
    a1 = jnp.dot(x, w1b, preferred_element_type=jnp.float32)
    a3 = jnp.dot(x, w3b, preferred_element_type=jnp.float32)
    h = (a1 / (1.0 + jnp.exp(-a1))) * a3
    acc_ref[...] += jnp.dot(h.astype(jnp.bfloat16),
                            w2_ref[0].astype(jnp.bfloat16),
                            preferred_element_type=jnp.float32)

    @pl.when(f == NF - 1)
    def _combine():
        eidx = lax.broadcasted_iota(jnp.int32, (B, L), 1)
        wcol = jnp.sum(jnp.where(eidx == e, we_ref[...], 0.0), axis=1,
                       keepdims=True)
        out_ref[...] += acc_ref[...] * wcol


def _moe_tc(x, we, w1, w3, w2):
    return pl.pallas_call(
        _moe_body,
        grid=(E, NF),
        in_specs=[
            pl.BlockSpec((B, D), lambda e, f: (0, 0)),
            pl.BlockSpec((B, L), lambda e, f: (0, 0)),
            pl.BlockSpec((1, D, FB), lambda e, f: (e, 0, f)),
            pl.BlockSpec((1, D, FB), lambda e, f: (e, 0, f)),
            pl.BlockSpec((1, FB, D), lambda e, f: (e, f, 0)),
        ],
        out_specs=pl.BlockSpec((B, D), lambda e, f: (0, 0)),
        out_shape=jax.ShapeDtypeStruct((B, D), jnp.float32),
        scratch_shapes=[
            pltpu.VMEM((B, D), jnp.float32),
        ],
        compiler_params=pltpu.CompilerParams(
            dimension_semantics=("arbitrary", "arbitrary"),
        ),
    )(x, we, w1, w3, w2)


@jax.jit
def kernel(x, gate_w, w1, w3, w2):
    g16 = gate_w.reshape(D // 2 * L)
    we = _sc_routing(x, g16)
    return _moe_tc(x, we, w1, w3, w2)


# speedup vs baseline: 1.0249x; 1.0249x over previous
import functools

import jax
import jax.numpy as jnp
from jax import lax
from jax.experimental import pallas as pl
from jax.experimental.pallas import tpu as pltpu
from jax.experimental.pallas import tpu_sc as plsc

E = 8
K = 2
B = 32
D = 4096
F = 2048

FB = 256
NF = F // FB

L = 16
NC = 2
NS = 16

_NEG = -1e30


def _sc_routing_body(lg_hbm, out_hbm, lv_ref, wv, sem):
    b = lax.axis_index("s") * NC + lax.axis_index("c")
    pltpu.sync_copy(lg_hbm.at[b], lv_ref)

    lane = lax.broadcasted_iota(jnp.int32, (L,), 0)
    lv = lv_ref[...]
    m1 = jnp.broadcast_to(jnp.max(lv, axis=0), (L,))
    i1 = jnp.broadcast_to(jnp.min(jnp.where(lv == m1, lane, L), axis=0), (L,))
    lv2 = jnp.where(lane == i1, _NEG, lv)
    m2 = jnp.broadcast_to(jnp.max(lv2, axis=0), (L,))
    i2 = jnp.broadcast_to(jnp.min(jnp.where(lv2 == m2, lane, L), axis=0), (L,))

    t = jnp.exp(m2 - m1)
    p1 = 1.0 / (1.0 + t)
    p2 = t / (1.0 + t)
    wv[...] = jnp.where(lane == i1, p1, 0.0) + jnp.where(lane == i2, p2, 0.0)
    pltpu.sync_copy(wv, out_hbm.at[b])


@functools.partial(
    pl.kernel,
    out_type=jax.ShapeDtypeStruct((B, L), jnp.float32),
    mesh=plsc.VectorSubcoreMesh(core_axis_name="c", subcore_axis_name="s"),
    compiler_params=pltpu.CompilerParams(needs_layout_passes=False),
    scratch_types=[
        pltpu.VMEM((L,), jnp.float32),
        pltpu.VMEM((L,), jnp.float32),
        pltpu.SemaphoreType.DMA,
    ],
)
def _sc_routing(lg_hbm, out_hbm, lv_ref, wv, sem):
    _sc_routing_body(lg_hbm, out_hbm, lv_ref, wv, sem)


def _gate_body(x_ref, gw_ref, out_ref):
    lg = jnp.dot(x_ref[...], gw_ref[...], preferred_element_type=jnp.float32)
    pad = jnp.full((B, L - E), _NEG, jnp.float32)
    out_ref[...] = jnp.concatenate([lg, pad], axis=1)


def _gate_tc(x, gate_w):
    return pl.pallas_call(
        _gate_body,
        out_shape=jax.ShapeDtypeStruct((B, L), jnp.float32),
    )(x, gate_w)


def _moe_body(x_ref, we_ref, w1_ref, w3_ref, w2_ref, out_ref, acc_ref):
    e = pl.program_id(0)
    f = pl.program_id(1)

    @pl.when((e == 0) & (f == 0))
    def _init():
        out_ref[...] = jnp.zeros_like(out_ref)

    @pl.when(f == 0)
    def _zero_acc():
        acc_ref[...] = jnp.zeros_like(acc_ref)

    x = x_ref[...].astype(jnp.bfloat16)
    w1b = w1_ref[0].astype(jnp.bfloat16)
    w3b = w3_ref[0].astype(jnp.bfloat16)
    a1 = jnp.dot(x, w1b, preferred_element_type=jnp.float32)
    a3 = jnp.dot(x, w3b, preferred_element_type=jnp.float32)
    h = (a1 / (1.0 + jnp.exp(-a1))) * a3
    acc_ref[...] += jnp.dot(h.astype(jnp.bfloat16),
                            w2_ref[0].astype(jnp.bfloat16),
                            preferred_element_type=jnp.float32)

    @pl.when(f == NF - 1)
    def _combine():
        eidx = lax.broadcasted_iota(jnp.int32, (B, L), 1)
        wcol = jnp.sum(jnp.where(eidx == e, we_ref[...], 0.0), axis=1,
                       keepdims=True)
        out_ref[...] += acc_ref[...] * wcol


def _moe_tc(x, we, w1, w3, w2):
    return pl.pallas_call(
        _moe_body,
        grid=(E, NF),
        in_specs=[
            pl.BlockSpec((B, D), lambda e, f: (0, 0)),
            pl.BlockSpec((B, L), lambda e, f: (0, 0)),
            pl.BlockSpec((1, D, FB), lambda e, f: (e, 0, f)),
            pl.BlockSpec((1, D, FB), lambda e, f: (e, 0, f)),
            pl.BlockSpec((1, FB, D), lambda e, f: (e, f, 0)),
        ],
        out_specs=pl.BlockSpec((B, D), lambda e, f: (0, 0)),
        out_shape=jax.ShapeDtypeStruct((B, D), jnp.float32),
        scratch_shapes=[
            pltpu.VMEM((B, D), jnp.float32),
        ],
        compiler_params=pltpu.CompilerParams(
            dimension_semantics=("arbitrary", "arbitrary"),
        ),
    )(x, we, w1, w3, w2)


@jax.jit
def kernel(x, gate_w, w1, w3, w2):
    logits = _gate_tc(x, gate_w)
    we = _sc_routing(logits)
    return _moe_tc(x, we, w1, w3, w2)
